# baseline (speedup 1.0000x reference)
import functools

import jax
import jax.numpy as jnp
from jax import lax
from jax.experimental import pallas as pl
from jax.experimental.pallas import tpu as pltpu
from jax.experimental.pallas import tpu_sc as plsc

IH, IW = 256, 704
NPIX = IH * IW
RS = NPIX // 2
M = 240000
MROWS = 1875
SENT = 0x3FFFFFFF
CH = 2048
NFULL = 117
TAIL = M - NFULL * CH
NCANVAS = 12
NTASK = NCANVAS * 2
UNROLL = 8

_mesh = plsc.VectorSubcoreMesh(core_axis_name="c", subcore_axis_name="s")


def _pid_body(py_ref, px_ref, pid_ref):
    py = py_ref[0]
    px = px_ref[0]
    on = (py < 256.0) & (py >= 0.0) & (px < 704.0) & (px >= 0.0)
    yi = py.astype(jnp.int32)
    xi = px.astype(jnp.int32)
    pid_ref[0] = jnp.where(on, yi * IW + xi, SENT)


def _pidify(pyf, pxf):
    return pl.pallas_call(
        _pid_body,
        grid=(NCANVAS,),
        in_specs=[
            pl.BlockSpec((1, MROWS, 128), lambda i: (i, 0, 0)),
            pl.BlockSpec((1, MROWS, 128), lambda i: (i, 0, 0)),
        ],
        out_specs=pl.BlockSpec((1, MROWS, 128), lambda i: (i, 0, 0)),
        out_shape=jax.ShapeDtypeStruct((NCANVAS, MROWS, 128), jnp.int32),
    )(pyf.reshape(NCANVAS, MROWS, 128), pxf.reshape(NCANVAS, MROWS, 128))


@functools.partial(
    pl.kernel,
    out_type=jax.ShapeDtypeStruct((NCANVAS * NPIX,), jnp.float32),
    mesh=_mesh,
    scratch_types=[
        pltpu.VMEM((CH,), jnp.int32),
        pltpu.VMEM((CH,), jnp.float32),
        pltpu.VMEM((CH,), jnp.int32),
        pltpu.VMEM((CH,), jnp.float32),
        pltpu.VMEM((RS,), jnp.float32),
        pltpu.SemaphoreType.DMA,
        pltpu.SemaphoreType.DMA,
    ],
    compiler_params=pltpu.CompilerParams(needs_layout_passes=False),
)
def _sc_scatter(pidh, dh, out, p0, d0, p1, d1, canvas, sem0, sem1):
    wid = lax.axis_index("s") * 2 + lax.axis_index("c")
    task = lax.rem(wid, NTASK)
    c = task // 2
    r = lax.rem(task, 2)
    lo = r * RS
    base = c * M

    def start(g, bp, bd, sem):
        pltpu.async_copy(pidh.at[pl.ds(base + g * CH, CH)], bp, sem)
        pltpu.async_copy(dh.at[pl.ds(base + g * CH, CH)], bd, sem)

    def start_tail(bp, bd, sem):
        pltpu.async_copy(pidh.at[pl.ds(base + NFULL * CH, TAIL)], bp.at[pl.ds(0, TAIL)], sem)
        pltpu.async_copy(dh.at[pl.ds(base + NFULL * CH, TAIL)], bd.at[pl.ds(0, TAIL)], sem)

    def wait(bp, bd, sem):
        pltpu.make_async_copy(pidh.at[pl.ds(0, CH)], bp, sem).wait()
        pltpu.make_async_copy(dh.at[pl.ds(0, CH)], bd, sem).wait()

    def wait_tail(bp, bd, sem):
        pltpu.make_async_copy(pidh.at[pl.ds(0, TAIL)], bp.at[pl.ds(0, TAIL)], sem).wait()
        pltpu.make_async_copy(dh.at[pl.ds(0, TAIL)], bd.at[pl.ds(0, TAIL)], sem).wait()

    def process(bp, bd, nvec):
        def vbody(i, carry):
            for k in range(UNROLL):
                off = i * (16 * UNROLL) + k * 16
                pid = bp[pl.ds(off, 16)]
                dv = bd[pl.ds(off, 16)]
                idx = pid - lo
                m = (idx >= 0) & (idx < RS)
                plsc.store_scatter(canvas, [idx], dv, mask=m)
            return carry
        lax.fori_loop(0, nvec // UNROLL, vbody, 0)

    start(0, p0, d0, sem0)
    start(1, p1, d1, sem1)

    def zbody(i, carry):
        canvas[pl.ds(i * 16, 16)] = jnp.zeros((16,), jnp.float32)
        return carry
    lax.fori_loop(0, RS // 16, zbody, 0)

    def chunk_pair(g2, carry):
        wait(p0, d0, sem0)
        process(p0, d0, CH // 16)

        @pl.when(g2 < 57)
        def _():
            start(2 * g2 + 2, p0, d0, sem0)

        @pl.when(g2 == 57)
        def _():
            start(116, p0, d0, sem0)

        wait(p1, d1, sem1)
        process(p1, d1, CH // 16)

        @pl.when(g2 < 57)
        def _():
            start(2 * g2 + 3, p1, d1, sem1)

        @pl.when(g2 == 57)
        def _():
            start_tail(p1, d1, sem1)
        return carry
    lax.fori_loop(0, 58, chunk_pair, 0)

    wait(p0, d0, sem0)
    process(p0, d0, CH // 16)
    wait_tail(p1, d1, sem1)
    process(p1, d1, TAIL // 16)

    pltpu.sync_copy(canvas, out.at[pl.ds(c * NPIX + lo, RS)])


def kernel(img, points, lidar2image, cam_intrinsic, camera2lidar, img_aug_matrix, lidar_aug_matrix):
    B = points.shape[0]
    N = lidar2image.shape[1]
    P = points.shape[1]
    heights = jnp.arange(0.25, 2.25, 0.25, dtype=jnp.float32)
    x = jnp.repeat(points[:, :, 0], 8, axis=1)
    y = jnp.repeat(points[:, :, 1], 8, axis=1)
    z = jnp.broadcast_to(jnp.tile(heights, P)[None, :], (B, M))
    tl = lidar_aug_matrix[:, :3, 3]
    cur3 = jnp.stack([x - tl[:, 0, None], y - tl[:, 1, None], z - tl[:, 2, None]], axis=1)
    G = jnp.linalg.inv(lidar_aug_matrix[:, :3, :3])
    c3 = jnp.matmul(G, cur3)
    v = jnp.matmul(lidar2image[:, :, :3, :3], c3[:, None])
    v = v + lidar2image[:, :, :3, 3][..., None]
    zc = jnp.clip(v[:, :, 2, :], 1e-05, 100000.0)
    u01 = v[:, :, :2, :] / zc[:, :, None, :]
    mid = jnp.concatenate([u01, zc[:, :, None, :]], axis=2)
    fin = jnp.matmul(img_aug_matrix[:, :, :3, :3], mid)
    fin = fin + img_aug_matrix[:, :, :3, 3][..., None]
    pyf = fin[:, :, 1, :].reshape(NCANVAS * M)
    pxf = fin[:, :, 0, :].reshape(NCANVAS * M)
    df = zc.reshape(NCANVAS * M)
    pid = _pidify(pyf, pxf)
    out = _sc_scatter(pid.reshape(-1), df)
    return out.reshape(B, N, 1, IH, IW)


# speedup vs baseline: 1.0083x; 1.0083x over previous
import functools

import jax
import jax.numpy as jnp
from jax import lax
from jax.experimental import pallas as pl
from jax.experimental.pallas import tpu as pltpu
from jax.experimental.pallas import tpu_sc as plsc

IH, IW = 256, 704
NPIX = IH * IW
RS = NPIX // 2
M = 240000
MROWS = 1875
SENT = 0x3FFFFFFF
CH = 2048
NFULL = 117
TAIL = M - NFULL * CH
NCANVAS = 12
NTASK = NCANVAS * 2
UNROLL = 8

_mesh = plsc.VectorSubcoreMesh(core_axis_name="c", subcore_axis_name="s")


def _pid_body(py_ref, px_ref, pid_ref):
    py = py_ref[0]
    px = px_ref[0]
    on = (py < 256.0) & (py >= 0.0) & (px < 704.0) & (px >= 0.0)
    yi = py.astype(jnp.int32)
    xi = px.astype(jnp.int32)
    pid_ref[0] = jnp.where(on, yi * IW + xi, SENT)


def _pidify(pyf, pxf):
    return pl.pallas_call(
        _pid_body,
        grid=(NCANVAS,),
        in_specs=[
            pl.BlockSpec((1, MROWS, 128), lambda i: (i, 0, 0)),
            pl.BlockSpec((1, MROWS, 128), lambda i: (i, 0, 0)),
        ],
        out_specs=pl.BlockSpec((1, MROWS, 128), lambda i: (i, 0, 0)),
        out_shape=jax.ShapeDtypeStruct((NCANVAS, MROWS, 128), jnp.int32),
    )(pyf.reshape(NCANVAS, MROWS, 128), pxf.reshape(NCANVAS, MROWS, 128))


@functools.partial(
    pl.kernel,
    out_type=jax.ShapeDtypeStruct((NCANVAS * NPIX,), jnp.float32),
    mesh=_mesh,
    scratch_types=[
        pltpu.VMEM((CH,), jnp.int32),
        pltpu.VMEM((CH,), jnp.float32),
        pltpu.VMEM((CH,), jnp.int32),
        pltpu.VMEM((CH,), jnp.float32),
        pltpu.VMEM((RS + 16,), jnp.float32),
        pltpu.SemaphoreType.DMA,
        pltpu.SemaphoreType.DMA,
    ],
    compiler_params=pltpu.CompilerParams(needs_layout_passes=False),
)
def _sc_scatter(pidh, dh, out, p0, d0, p1, d1, canvas, sem0, sem1):
    wid = lax.axis_index("s") * 2 + lax.axis_index("c")
    task = lax.rem(wid, NTASK)
    c = task // 2
    r = lax.rem(task, 2)
    lo = r * RS
    base = c * M

    def start(g, bp, bd, sem):
        pltpu.async_copy(pidh.at[pl.ds(base + g * CH, CH)], bp, sem)
        pltpu.async_copy(dh.at[pl.ds(base + g * CH, CH)], bd, sem)

    def start_tail(bp, bd, sem):
        pltpu.async_copy(pidh.at[pl.ds(base + NFULL * CH, TAIL)], bp.at[pl.ds(0, TAIL)], sem)
        pltpu.async_copy(dh.at[pl.ds(base + NFULL * CH, TAIL)], bd.at[pl.ds(0, TAIL)], sem)

    def wait(bp, bd, sem):
        pltpu.make_async_copy(pidh.at[pl.ds(0, CH)], bp, sem).wait()
        pltpu.make_async_copy(dh.at[pl.ds(0, CH)], bd, sem).wait()

    def wait_tail(bp, bd, sem):
        pltpu.make_async_copy(pidh.at[pl.ds(0, TAIL)], bp.at[pl.ds(0, TAIL)], sem).wait()
        pltpu.make_async_copy(dh.at[pl.ds(0, TAIL)], bd.at[pl.ds(0, TAIL)], sem).wait()

    def process(bp, bd, nvec):
        def vbody(i, carry):
            for k in range(UNROLL):
                off = i * (16 * UNROLL) + k * 16
                pid = bp[pl.ds(off, 16)]
                dv = bd[pl.ds(off, 16)]
                idx = pid - lo
                idxc = plsc.bitcast(
                    jnp.minimum(plsc.bitcast(idx, jnp.uint32), jnp.uint32(RS)),
                    jnp.int32)
                plsc.store_scatter(canvas, [idxc], dv)
            return carry
        lax.fori_loop(0, nvec // UNROLL, vbody, 0)

    start(0, p0, d0, sem0)
    start(1, p1, d1, sem1)

    def zbody(i, carry):
        canvas[pl.ds(i * 16, 16)] = jnp.zeros((16,), jnp.float32)
        return carry
    lax.fori_loop(0, RS // 16 + 1, zbody, 0)

    def chunk_pair(g2, carry):
        wait(p0, d0, sem0)
        process(p0, d0, CH // 16)

        @pl.when(g2 < 57)
        def _():
            start(2 * g2 + 2, p0, d0, sem0)

        @pl.when(g2 == 57)
        def _():
            start(116, p0, d0, sem0)

        wait(p1, d1, sem1)
        process(p1, d1, CH // 16)

        @pl.when(g2 < 57)
        def _():
            start(2 * g2 + 3, p1, d1, sem1)

        @pl.when(g2 == 57)
        def _():
            start_tail(p1, d1, sem1)
        return carry
    lax.fori_loop(0, 58, chunk_pair, 0)

    wait(p0, d0, sem0)
    process(p0, d0, CH // 16)
    wait_tail(p1, d1, sem1)
    process(p1, d1, TAIL // 16)

    pltpu.sync_copy(canvas.at[pl.ds(0, RS)], out.at[pl.ds(c * NPIX + lo, RS)])


def kernel(img, points, lidar2image, cam_intrinsic, camera2lidar, img_aug_matrix, lidar_aug_matrix):
    B = points.shape[0]
    N = lidar2image.shape[1]
    P = points.shape[1]
    heights = jnp.arange(0.25, 2.25, 0.25, dtype=jnp.float32)
    x = jnp.repeat(points[:, :, 0], 8, axis=1)
    y = jnp.repeat(points[:, :, 1], 8, axis=1)
    z = jnp.broadcast_to(jnp.tile(heights, P)[None, :], (B, M))
    tl = lidar_aug_matrix[:, :3, 3]
    cur3 = jnp.stack([x - tl[:, 0, None], y - tl[:, 1, None], z - tl[:, 2, None]], axis=1)
    G = jnp.linalg.inv(lidar_aug_matrix[:, :3, :3])
    c3 = jnp.matmul(G, cur3)
    v = jnp.matmul(lidar2image[:, :, :3, :3], c3[:, None])
    v = v + lidar2image[:, :, :3, 3][..., None]
    zc = jnp.clip(v[:, :, 2, :], 1e-05, 100000.0)
    u01 = v[:, :, :2, :] / zc[:, :, None, :]
    mid = jnp.concatenate([u01, zc[:, :, None, :]], axis=2)
    fin = jnp.matmul(img_aug_matrix[:, :, :3, :3], mid)
    fin = fin + img_aug_matrix[:, :, :3, 3][..., None]
    pyf = fin[:, :, 1, :].reshape(NCANVAS * M)
    pxf = fin[:, :, 0, :].reshape(NCANVAS * M)
    df = zc.reshape(NCANVAS * M)
    pid = _pidify(pyf, pxf)
    out = _sc_scatter(pid.reshape(-1), df)
    return out.reshape(B, N, 1, IH, IW)
